# baseline (speedup 1.0000x reference)
import functools

import jax
import jax.numpy as jnp
from jax import lax
from jax.experimental import pallas as pl
from jax.experimental.pallas import tpu as pltpu
from jax.experimental.pallas import tpu_sc as plsc

N_NODES = 10000
D_FEAT = 128
N_EDGES = 320000

NC = 2
NS = 16
NW = NC * NS
FPT = D_FEAT // NW
FLAT = N_NODES * FPT
FPAD = 40064
FROW = FPAD // 128
EB = 2048
EROW = 16
NBLK = 160
E_PAD = NBLK * EB
GPR = 8

_mesh = plsc.VectorSubcoreMesh(core_axis_name="c", subcore_axis_name="s")


@functools.partial(
    pl.kernel,
    out_type=jax.ShapeDtypeStruct((NC, NS, FROW, 128), jnp.float32),
    mesh=_mesh,
    compiler_params=pltpu.CompilerParams(needs_layout_passes=False),
    scratch_types=[
        pltpu.VMEM((FROW, 128), jnp.float32),
        pltpu.VMEM((FROW, 128), jnp.float32),
        pltpu.VMEM((EROW, 128), jnp.int32),
        pltpu.VMEM((EROW, 128), jnp.int32),
        pltpu.VMEM((EROW, 128), jnp.int32),
        pltpu.VMEM((EROW, 128), jnp.int32),
        pltpu.VMEM((EROW, 128), jnp.float32),
        pltpu.VMEM((EROW, 128), jnp.float32),
        pltpu.SemaphoreType.DMA,
        pltpu.SemaphoreType.DMA,
    ],
)
def _mp_sc_kernel(xt_hbm, idxj_hbm, idxi_hbm, val_hbm, out_hbm,
                  xloc, acc, jb0, jb1, ib0, ib1, vb0, vb1, sem0, sem1):
    c = lax.axis_index("c")
    s = lax.axis_index("s")
    jbs = (jb0, jb1)
    ibs = (ib0, ib1)
    vbs = (vb0, vb1)
    sems = (sem0, sem1)

    pltpu.sync_copy(xt_hbm.at[c].at[s], xloc)
    zeros16 = jnp.zeros((16,), jnp.float32)

    def zero_row(r, carry):
        for q in range(8):
            acc[r, pl.ds(q * 16, 16)] = zeros16
        return carry

    lax.fori_loop(0, FROW, zero_row, 0)

    def issue_block(b, u):
        pltpu.async_copy(idxj_hbm.at[b], jbs[u], sems[u])
        pltpu.async_copy(idxi_hbm.at[b], ibs[u], sems[u])
        pltpu.async_copy(val_hbm.at[b], vbs[u], sems[u])

    def wait_block(u):
        pltpu.make_async_copy(idxj_hbm.at[0], jbs[u], sems[u]).wait()
        pltpu.make_async_copy(idxi_hbm.at[0], ibs[u], sems[u]).wait()
        pltpu.make_async_copy(val_hbm.at[0], vbs[u], sems[u]).wait()

    issue_block(0, 0)
    issue_block(1, 1)

    def process_block(u):
        jb, ib, vb = jbs[u], ibs[u], vbs[u]

        def row_body(r, carry):
            for g in range(GPR):
                j16 = jb[r, pl.ds(g * 16, 16)]
                i16 = ib[r, pl.ds(g * 16, 16)]
                v16 = vb[r, pl.ds(g * 16, 16)]
                jflat = j16 * FPT
                iflat = i16 * FPT
                for q in range(FPT):
                    jq = jflat + q
                    iq = iflat + q
                    feat = plsc.load_gather(
                        xloc, [lax.shift_right_logical(jq, 7),
                               lax.bitwise_and(jq, 127)])
                    prod = feat * v16
                    plsc.addupdate_scatter(
                        acc, [lax.shift_right_logical(iq, 7),
                              lax.bitwise_and(iq, 127)], prod)
            return carry

        lax.fori_loop(0, EROW, row_body, 0)

    def pair_body(p, carry):
        for u in range(2):
            wait_block(u)
            process_block(u)
            issue_block(2 * p + u + 2, u)
        return carry

    lax.fori_loop(0, NBLK // 2, pair_body, 0)
    wait_block(0)
    wait_block(1)

    pltpu.sync_copy(acc, out_hbm.at[c].at[s])


def kernel(x, a_indices, a_values):
    pad = E_PAD - N_EDGES
    xtra = 2 * EB
    idx_i = jnp.pad(a_indices[0].astype(jnp.int32), (0, pad + xtra))
    idx_j = jnp.pad(a_indices[1].astype(jnp.int32), (0, pad + xtra))
    vals = jnp.pad(a_values.astype(jnp.float32), (0, pad + xtra))
    idx_i = idx_i.reshape(NBLK + 2, EROW, 128)
    idx_j = idx_j.reshape(NBLK + 2, EROW, 128)
    vals = vals.reshape(NBLK + 2, EROW, 128)
    xt = x.reshape(N_NODES, NW, FPT).transpose(1, 0, 2).reshape(NW, FLAT)
    xt = jnp.pad(xt, ((0, 0), (0, FPAD - FLAT)))
    xt = xt.reshape(NC, NS, FROW, 128)
    out_t = _mp_sc_kernel(xt, idx_j, idx_i, vals)
    out_t = out_t.reshape(NW, FPAD)[:, :FLAT]
    return out_t.reshape(NW, N_NODES, FPT).transpose(1, 0, 2).reshape(
        N_NODES, D_FEAT)
